# baseline (speedup 1.0000x reference)
import functools

import jax
import jax.numpy as jnp
from jax import lax
from jax.experimental import pallas as pl
from jax.experimental.pallas import tpu as pltpu
from jax.experimental.pallas import tpu_sc as plsc

N = 10000
D = 128
E = 320000
NC = 2
NS = 16
NW = NC * NS
CH = 128
NCH = 80
EPT = CH * NCH
E_PAD = NW * EPT
NACC = 10240
RPT = NACC // NS
RB = 1000
IG = 16
NCH0 = 48
NCH1 = 112
NCHX = max(NCH0, NCH1)

_mesh = plsc.VectorSubcoreMesh(core_axis_name="c", subcore_axis_name="s")


@functools.partial(
    pl.kernel,
    out_type=jax.ShapeDtypeStruct((NC, NACC, D), jnp.float32),
    mesh=_mesh,
    scratch_types=[
        pltpu.VMEM((NCH, CH), jnp.int32),
        pltpu.VMEM((CH, D), jnp.float32),
        pltpu.VMEM_SHARED((NACC, D), jnp.float32),
        pltpu.SemaphoreType.DMA,
    ],
)
def _deg_kernel(dst_hbm, ones_hbm, zeros_hbm, out_hbm, dst_v, ones_v, acc, sem):
    c = lax.axis_index("c")
    s = lax.axis_index("s")
    wid = c * NS + s
    pltpu.sync_copy(zeros_hbm, acc.at[pl.ds(s * RPT, RPT)])
    pltpu.sync_copy(ones_hbm, ones_v)
    pltpu.sync_copy(dst_hbm.at[wid], dst_v)
    plsc.subcore_barrier()

    def body(j, carry):
        pltpu.sync_copy(ones_v, acc.at[dst_v.at[j]], add=True)
        return carry

    lax.fori_loop(0, NCH, body, 0)
    plsc.subcore_barrier()
    pltpu.sync_copy(acc.at[pl.ds(s * RPT, RPT)],
                    out_hbm.at[c].at[pl.ds(s * RPT, RPT)])


@functools.partial(
    pl.kernel,
    out_type=jax.ShapeDtypeStruct((NC, NACC, D), jnp.float32),
    mesh=_mesh,
    scratch_types=[pltpu.VMEM((IG, CH), jnp.int32)] * 4
      + [pltpu.VMEM((CH, D), jnp.float32)] * 2
      + [pltpu.VMEM_SHARED((NACC, D), jnp.float32)]
      + [pltpu.SemaphoreType.DMA] * 6,
)
def _msg_kernel(h_hbm, src_hbm, dst_hbm, zeros_hbm, out_hbm,
                sidx0, sidx1, didx0, didx1, buf0, buf1, acc,
                isem0, isem1, gsem0, gsem1, ssem0, ssem1):
    sidxs = (sidx0, sidx1)
    didxs = (didx0, didx1)
    bufs = (buf0, buf1)
    isems = (isem0, isem1)
    gsems = (gsem0, gsem1)
    ssems = (ssem0, ssem1)
    c = lax.axis_index("c")
    s = lax.axis_index("s")
    wid = c * NS + s

    def idx_load(g):
        sl = g % 2
        pltpu.async_copy(src_hbm.at[wid].at[pl.ds(g * IG, IG)],
                         sidxs[sl], isems[sl])
        pltpu.async_copy(dst_hbm.at[wid].at[pl.ds(g * IG, IG)],
                         didxs[sl], isems[sl])

    def idx_wait(g):
        sl = g % 2
        pltpu.make_async_copy(src_hbm.at[wid].at[pl.ds(g * IG, IG)],
                              sidxs[sl], isems[sl]).wait()
        pltpu.make_async_copy(dst_hbm.at[wid].at[pl.ds(g * IG, IG)],
                              didxs[sl], isems[sl]).wait()

    def gather(g, jj, b):
        pltpu.async_copy(h_hbm.at[sidxs[g % 2].at[jj]], bufs[b], gsems[b])

    def gather_wait(g, jj, b):
        pltpu.make_async_copy(h_hbm.at[sidxs[g % 2].at[jj]],
                              bufs[b], gsems[b]).wait()

    def scatter(g, jj, b):
        pltpu.async_copy(bufs[b], acc.at[didxs[g % 2].at[jj]], ssems[b],
                         add=True)

    def scatter_wait(g, jj, b):
        pltpu.make_async_copy(bufs[b], acc.at[didxs[g % 2].at[jj]],
                              ssems[b]).wait()

    pltpu.sync_copy(zeros_hbm, acc.at[pl.ds(s * RPT, RPT)])
    idx_load(0)
    plsc.subcore_barrier()

    def pipeline(ng):
        idx_wait(0)
        for g in range(ng):
            gather(g, 0, 0)
            gather(g, 1, 1)
            if g + 1 < ng:
                idx_load(g + 1)

            def inner(jj2, carry):
                jj = jj2 * 2
                gather_wait(g, jj, 0)
                scatter(g, jj, 0)
                gather_wait(g, jj + 1, 1)
                scatter(g, jj + 1, 1)
                scatter_wait(g, jj, 0)
                gather(g, jj + 2, 0)
                scatter_wait(g, jj + 1, 1)
                gather(g, jj + 3, 1)
                return carry

            lax.fori_loop(0, IG // 2 - 1, inner, 0)
            gather_wait(g, IG - 2, 0)
            scatter(g, IG - 2, 0)
            gather_wait(g, IG - 1, 1)
            scatter(g, IG - 1, 1)
            scatter_wait(g, IG - 2, 0)
            scatter_wait(g, IG - 1, 1)
            if g + 1 < ng:
                idx_wait(g + 1)

    @pl.when(c == 0)
    def _():
        pipeline(NCH0 // IG)

    @pl.when(c == 1)
    def _():
        pipeline(NCH1 // IG)

    plsc.subcore_barrier()
    pltpu.sync_copy(acc.at[pl.ds(s * RPT, RPT)],
                    out_hbm.at[c].at[pl.ds(s * RPT, RPT)])


def _l1_body(x_ref, w1_ref, d0_ref, d1_ref, h1p_ref, dis_ref):
    deg = d0_ref[...][:, :1] + d1_ref[...][:, :1] + 1.0
    dis = lax.rsqrt(deg)
    dis_ref[...] = dis
    h = jnp.dot(x_ref[...], w1_ref[...], preferred_element_type=jnp.float32)
    h1p_ref[...] = h * dis


def _l2_body(p0_ref, p1_ref, h1p_ref, dis_ref, b1_ref, w2_ref, h2p_ref):
    dis = dis_ref[...]
    t = (p0_ref[...] + p1_ref[...] + h1p_ref[...]) * dis + b1_ref[...]
    h = jnp.maximum(t, 0.0)
    h2p_ref[...] = jnp.dot(h, w2_ref[...],
                           preferred_element_type=jnp.float32) * dis


def _out_body(q0_ref, q1_ref, h2p_ref, dis_ref, b2_ref, out_ref):
    t = (q0_ref[...] + q1_ref[...] + h2p_ref[...]) * dis_ref[...] + b2_ref[...]
    m = jnp.max(t, axis=1, keepdims=True)
    lse = jnp.log(jnp.sum(jnp.exp(t - m), axis=1, keepdims=True)) + m
    out_ref[...] = t - lse


_row_spec = pl.BlockSpec((RB, D), lambda i: (i, 0))
_col_spec = pl.BlockSpec((RB, 1), lambda i: (i, 0))
_deg_spec = pl.BlockSpec((RB, D), lambda i: (i, 0))
_w_spec = pl.BlockSpec((D, D), lambda i: (0, 0))
_b_spec = pl.BlockSpec((1, D), lambda i: (0, 0))

_l1_call = pl.pallas_call(
    _l1_body,
    grid=(N // RB,),
    in_specs=[_row_spec, _w_spec, _deg_spec, _deg_spec],
    out_specs=[_row_spec, _col_spec],
    out_shape=[jax.ShapeDtypeStruct((N, D), jnp.float32),
               jax.ShapeDtypeStruct((N, 1), jnp.float32)],
)

_l2_call = pl.pallas_call(
    _l2_body,
    grid=(N // RB,),
    in_specs=[_row_spec, _row_spec, _row_spec, _col_spec, _b_spec, _w_spec],
    out_specs=_row_spec,
    out_shape=jax.ShapeDtypeStruct((N, D), jnp.float32),
)

_out_call = pl.pallas_call(
    _out_body,
    grid=(N // RB,),
    in_specs=[_row_spec, _row_spec, _row_spec, _col_spec, _b_spec],
    out_specs=_row_spec,
    out_shape=jax.ShapeDtypeStruct((N, D), jnp.float32),
)


def kernel(x, edge_index, W1, b1, W2, b2):
    src = edge_index[0].astype(jnp.int32)
    dst = edge_index[1].astype(jnp.int32)
    pad = E_PAD - E
    pad_src = jnp.zeros((pad,), jnp.int32)
    pad_dst = N + (jnp.arange(pad, dtype=jnp.int32) % (NACC - N))
    srcp = jnp.concatenate([src, pad_src])
    dstp = jnp.concatenate([dst, pad_dst])
    dst3 = dstp.reshape(NW, NCH, CH)

    E0 = NS * NCH0 * CH

    def msg_layout(a):
        a0 = a[:E0].reshape(NS, NCH0, CH)
        a0 = jnp.pad(a0, ((0, 0), (0, NCHX - NCH0), (0, 0)))
        a1 = a[E0:].reshape(NS, NCH1, CH)
        a1 = jnp.pad(a1, ((0, 0), (0, NCHX - NCH1), (0, 0)))
        return jnp.concatenate([a0, a1], axis=0)

    srcm = msg_layout(srcp)
    dstm = msg_layout(dstp)
    onesD = jnp.ones((CH, D), jnp.float32)
    zerosD = jnp.zeros((RPT, D), jnp.float32)

    degD = _deg_kernel(dst3, onesD, zerosD)
    h1p, dis = _l1_call(x, W1, degD[0, :N], degD[1, :N])
    p = _msg_kernel(h1p, srcm, dstm, zerosD)
    h2p = _l2_call(p[0, :N], p[1, :N], h1p, dis, b1.reshape(1, D), W2)
    q = _msg_kernel(h2p, srcm, dstm, zerosD)
    return _out_call(q[0, :N], q[1, :N], h2p, dis, b2.reshape(1, D))


# speedup vs baseline: 1.0596x; 1.0596x over previous
import functools

import jax
import jax.numpy as jnp
from jax import lax
from jax.experimental import pallas as pl
from jax.experimental.pallas import tpu as pltpu
from jax.experimental.pallas import tpu_sc as plsc

N = 10000
D = 128
E = 320000
NC = 2
NS = 16
NW = NC * NS
CH = 128
NCH = 80
EPT = CH * NCH
E_PAD = NW * EPT
NACC = 10240
RPT = NACC // NS
RB = 1000
IG = 16
NCH0 = 112
NCH1 = 48
NCHX = max(NCH0, NCH1)

_mesh = plsc.VectorSubcoreMesh(core_axis_name="c", subcore_axis_name="s")


@functools.partial(
    pl.kernel,
    out_type=jax.ShapeDtypeStruct((NC, NACC, D), jnp.float32),
    mesh=_mesh,
    scratch_types=[
        pltpu.VMEM((NCH, CH), jnp.int32),
        pltpu.VMEM((CH, D), jnp.float32),
        pltpu.VMEM_SHARED((NACC, D), jnp.float32),
        pltpu.SemaphoreType.DMA,
    ],
)
def _deg_kernel(dst_hbm, ones_hbm, zeros_hbm, out_hbm, dst_v, ones_v, acc, sem):
    c = lax.axis_index("c")
    s = lax.axis_index("s")
    wid = c * NS + s
    pltpu.sync_copy(zeros_hbm, acc.at[pl.ds(s * RPT, RPT)])
    pltpu.sync_copy(ones_hbm, ones_v)
    pltpu.sync_copy(dst_hbm.at[wid], dst_v)
    plsc.subcore_barrier()

    def body(j, carry):
        pltpu.sync_copy(ones_v, acc.at[dst_v.at[j]], add=True)
        return carry

    lax.fori_loop(0, NCH, body, 0)
    plsc.subcore_barrier()
    pltpu.sync_copy(acc.at[pl.ds(s * RPT, RPT)],
                    out_hbm.at[c].at[pl.ds(s * RPT, RPT)])


@functools.partial(
    pl.kernel,
    out_type=jax.ShapeDtypeStruct((NC, NACC, D), jnp.float32),
    mesh=_mesh,
    scratch_types=[pltpu.VMEM((IG, CH), jnp.int32)] * 4
      + [pltpu.VMEM((CH, D), jnp.float32)] * 2
      + [pltpu.VMEM_SHARED((NACC, D), jnp.float32)]
      + [pltpu.SemaphoreType.DMA] * 6,
)
def _msg_kernel(h_hbm, src_hbm, dst_hbm, zeros_hbm, out_hbm,
                sidx0, sidx1, didx0, didx1, buf0, buf1, acc,
                isem0, isem1, gsem0, gsem1, ssem0, ssem1):
    sidxs = (sidx0, sidx1)
    didxs = (didx0, didx1)
    bufs = (buf0, buf1)
    isems = (isem0, isem1)
    gsems = (gsem0, gsem1)
    ssems = (ssem0, ssem1)
    c = lax.axis_index("c")
    s = lax.axis_index("s")
    wid = c * NS + s

    def idx_load(g):
        sl = g % 2
        pltpu.async_copy(src_hbm.at[wid].at[pl.ds(g * IG, IG)],
                         sidxs[sl], isems[sl])
        pltpu.async_copy(dst_hbm.at[wid].at[pl.ds(g * IG, IG)],
                         didxs[sl], isems[sl])

    def idx_wait(g):
        sl = g % 2
        pltpu.make_async_copy(src_hbm.at[wid].at[pl.ds(g * IG, IG)],
                              sidxs[sl], isems[sl]).wait()
        pltpu.make_async_copy(dst_hbm.at[wid].at[pl.ds(g * IG, IG)],
                              didxs[sl], isems[sl]).wait()

    def gather(g, jj, b):
        pltpu.async_copy(h_hbm.at[sidxs[g % 2].at[jj]], bufs[b], gsems[b])

    def gather_wait(g, jj, b):
        pltpu.make_async_copy(h_hbm.at[sidxs[g % 2].at[jj]],
                              bufs[b], gsems[b]).wait()

    def scatter(g, jj, b):
        pltpu.async_copy(bufs[b], acc.at[didxs[g % 2].at[jj]], ssems[b],
                         add=True)

    def scatter_wait(g, jj, b):
        pltpu.make_async_copy(bufs[b], acc.at[didxs[g % 2].at[jj]],
                              ssems[b]).wait()

    pltpu.sync_copy(zeros_hbm, acc.at[pl.ds(s * RPT, RPT)])
    idx_load(0)
    plsc.subcore_barrier()

    def pipeline(ng):
        idx_wait(0)
        for g in range(ng):
            gather(g, 0, 0)
            gather(g, 1, 1)
            if g + 1 < ng:
                idx_load(g + 1)

            def inner(jj2, carry):
                jj = jj2 * 2
                gather_wait(g, jj, 0)
                scatter(g, jj, 0)
                gather_wait(g, jj + 1, 1)
                scatter(g, jj + 1, 1)
                scatter_wait(g, jj, 0)
                gather(g, jj + 2, 0)
                scatter_wait(g, jj + 1, 1)
                gather(g, jj + 3, 1)
                return carry

            lax.fori_loop(0, IG // 2 - 1, inner, 0)
            gather_wait(g, IG - 2, 0)
            scatter(g, IG - 2, 0)
            gather_wait(g, IG - 1, 1)
            scatter(g, IG - 1, 1)
            scatter_wait(g, IG - 2, 0)
            scatter_wait(g, IG - 1, 1)
            if g + 1 < ng:
                idx_wait(g + 1)

    @pl.when(c == 0)
    def _():
        pipeline(NCH0 // IG)

    @pl.when(c == 1)
    def _():
        pipeline(NCH1 // IG)

    plsc.subcore_barrier()
    pltpu.sync_copy(acc.at[pl.ds(s * RPT, RPT)],
                    out_hbm.at[c].at[pl.ds(s * RPT, RPT)])


def _l1_body(x_ref, w1_ref, d0_ref, d1_ref, h1p_ref, dis_ref):
    deg = d0_ref[...][:, :1] + d1_ref[...][:, :1] + 1.0
    dis = lax.rsqrt(deg)
    dis_ref[...] = dis
    h = jnp.dot(x_ref[...], w1_ref[...], preferred_element_type=jnp.float32)
    h1p_ref[...] = h * dis


def _l2_body(p0_ref, p1_ref, h1p_ref, dis_ref, b1_ref, w2_ref, h2p_ref):
    dis = dis_ref[...]
    t = (p0_ref[...] + p1_ref[...] + h1p_ref[...]) * dis + b1_ref[...]
    h = jnp.maximum(t, 0.0)
    h2p_ref[...] = jnp.dot(h, w2_ref[...],
                           preferred_element_type=jnp.float32) * dis


def _out_body(q0_ref, q1_ref, h2p_ref, dis_ref, b2_ref, out_ref):
    t = (q0_ref[...] + q1_ref[...] + h2p_ref[...]) * dis_ref[...] + b2_ref[...]
    m = jnp.max(t, axis=1, keepdims=True)
    lse = jnp.log(jnp.sum(jnp.exp(t - m), axis=1, keepdims=True)) + m
    out_ref[...] = t - lse


_row_spec = pl.BlockSpec((RB, D), lambda i: (i, 0))
_col_spec = pl.BlockSpec((RB, 1), lambda i: (i, 0))
_deg_spec = pl.BlockSpec((RB, D), lambda i: (i, 0))
_w_spec = pl.BlockSpec((D, D), lambda i: (0, 0))
_b_spec = pl.BlockSpec((1, D), lambda i: (0, 0))

_l1_call = pl.pallas_call(
    _l1_body,
    grid=(N // RB,),
    in_specs=[_row_spec, _w_spec, _deg_spec, _deg_spec],
    out_specs=[_row_spec, _col_spec],
    out_shape=[jax.ShapeDtypeStruct((N, D), jnp.float32),
               jax.ShapeDtypeStruct((N, 1), jnp.float32)],
)

_l2_call = pl.pallas_call(
    _l2_body,
    grid=(N // RB,),
    in_specs=[_row_spec, _row_spec, _row_spec, _col_spec, _b_spec, _w_spec],
    out_specs=_row_spec,
    out_shape=jax.ShapeDtypeStruct((N, D), jnp.float32),
)

_out_call = pl.pallas_call(
    _out_body,
    grid=(N // RB,),
    in_specs=[_row_spec, _row_spec, _row_spec, _col_spec, _b_spec],
    out_specs=_row_spec,
    out_shape=jax.ShapeDtypeStruct((N, D), jnp.float32),
)


def kernel(x, edge_index, W1, b1, W2, b2):
    src = edge_index[0].astype(jnp.int32)
    dst = edge_index[1].astype(jnp.int32)
    pad = E_PAD - E
    pad_src = jnp.zeros((pad,), jnp.int32)
    pad_dst = N + (jnp.arange(pad, dtype=jnp.int32) % (NACC - N))
    srcp = jnp.concatenate([src, pad_src])
    dstp = jnp.concatenate([dst, pad_dst])
    dst3 = dstp.reshape(NW, NCH, CH)

    E0 = NS * NCH0 * CH

    def msg_layout(a):
        a0 = a[:E0].reshape(NS, NCH0, CH)
        a0 = jnp.pad(a0, ((0, 0), (0, NCHX - NCH0), (0, 0)))
        a1 = a[E0:].reshape(NS, NCH1, CH)
        a1 = jnp.pad(a1, ((0, 0), (0, NCHX - NCH1), (0, 0)))
        return jnp.concatenate([a0, a1], axis=0)

    srcm = msg_layout(srcp)
    dstm = msg_layout(dstp)
    onesD = jnp.ones((CH, D), jnp.float32)
    zerosD = jnp.zeros((RPT, D), jnp.float32)

    degD = _deg_kernel(dst3, onesD, zerosD)
    h1p, dis = _l1_call(x, W1, degD[0, :N], degD[1, :N])
    p = _msg_kernel(h1p, srcm, dstm, zerosD)
    h2p = _l2_call(p[0, :N], p[1, :N], h1p, dis, b1.reshape(1, D), W2)
    q = _msg_kernel(h2p, srcm, dstm, zerosD)
    return _out_call(q[0, :N], q[1, :N], h2p, dis, b2.reshape(1, D))


# speedup vs baseline: 1.1676x; 1.1020x over previous
import functools

import jax
import jax.numpy as jnp
from jax import lax
from jax.experimental import pallas as pl
from jax.experimental.pallas import tpu as pltpu
from jax.experimental.pallas import tpu_sc as plsc

N = 10000
D = 128
E = 320000
NC = 2
NS = 16
NW = NC * NS
CH = 128
NCH = 80
EPT = CH * NCH
E_PAD = NW * EPT
NACC = 10240
RPT = NACC // NS
RB = 1000
IG = 16
NCH0 = 80
NCH1 = 80
NCHX = max(NCH0, NCH1)

_mesh = plsc.VectorSubcoreMesh(core_axis_name="c", subcore_axis_name="s")


@functools.partial(
    pl.kernel,
    out_type=jax.ShapeDtypeStruct((NC, NACC, D), jnp.float32),
    mesh=_mesh,
    scratch_types=[
        pltpu.VMEM((NCH, CH), jnp.int32),
        pltpu.VMEM((CH, D), jnp.float32),
        pltpu.VMEM_SHARED((NACC, D), jnp.float32),
        pltpu.SemaphoreType.DMA,
    ],
)
def _deg_kernel(dst_hbm, ones_hbm, zeros_hbm, out_hbm, dst_v, ones_v, acc, sem):
    c = lax.axis_index("c")
    s = lax.axis_index("s")
    wid = c * NS + s
    pltpu.sync_copy(zeros_hbm, acc.at[pl.ds(s * RPT, RPT)])
    pltpu.sync_copy(ones_hbm, ones_v)
    pltpu.sync_copy(dst_hbm.at[wid], dst_v)
    plsc.subcore_barrier()

    def body(j, carry):
        pltpu.sync_copy(ones_v, acc.at[dst_v.at[j]], add=True)
        return carry

    lax.fori_loop(0, NCH, body, 0)
    plsc.subcore_barrier()
    pltpu.sync_copy(acc.at[pl.ds(s * RPT, RPT)],
                    out_hbm.at[c].at[pl.ds(s * RPT, RPT)])


@functools.partial(
    pl.kernel,
    out_type=jax.ShapeDtypeStruct((NC, NACC, D), jnp.float32),
    mesh=_mesh,
    scratch_types=[pltpu.VMEM((IG, CH), jnp.int32)] * 4
      + [pltpu.VMEM((CH, D), jnp.float32)] * 2
      + [pltpu.VMEM_SHARED((NACC, D), jnp.float32)]
      + [pltpu.SemaphoreType.DMA] * 6,
)
def _msg_kernel(h_hbm, src_hbm, dst_hbm, zeros_hbm, out_hbm,
                sidx0, sidx1, didx0, didx1, buf0, buf1, acc,
                isem0, isem1, gsem0, gsem1, ssem0, ssem1):
    sidxs = (sidx0, sidx1)
    didxs = (didx0, didx1)
    bufs = (buf0, buf1)
    isems = (isem0, isem1)
    gsems = (gsem0, gsem1)
    ssems = (ssem0, ssem1)
    c = lax.axis_index("c")
    s = lax.axis_index("s")
    wid = c * NS + s

    def idx_load(g):
        sl = g % 2
        pltpu.async_copy(src_hbm.at[wid].at[pl.ds(g * IG, IG)],
                         sidxs[sl], isems[sl])
        pltpu.async_copy(dst_hbm.at[wid].at[pl.ds(g * IG, IG)],
                         didxs[sl], isems[sl])

    def idx_wait(g):
        sl = g % 2
        pltpu.make_async_copy(src_hbm.at[wid].at[pl.ds(g * IG, IG)],
                              sidxs[sl], isems[sl]).wait()
        pltpu.make_async_copy(dst_hbm.at[wid].at[pl.ds(g * IG, IG)],
                              didxs[sl], isems[sl]).wait()

    def gather(g, jj, b):
        pltpu.async_copy(h_hbm.at[sidxs[g % 2].at[jj]], bufs[b], gsems[b])

    def gather_wait(g, jj, b):
        pltpu.make_async_copy(h_hbm.at[sidxs[g % 2].at[jj]],
                              bufs[b], gsems[b]).wait()

    def scatter(g, jj, b):
        pltpu.async_copy(bufs[b], acc.at[didxs[g % 2].at[jj]], ssems[b],
                         add=True)

    def scatter_wait(g, jj, b):
        pltpu.make_async_copy(bufs[b], acc.at[didxs[g % 2].at[jj]],
                              ssems[b]).wait()

    pltpu.sync_copy(zeros_hbm, acc.at[pl.ds(s * RPT, RPT)])
    idx_load(0)
    plsc.subcore_barrier()

    def pipeline(ng):
        idx_wait(0)
        for g in range(ng):
            gather(g, 0, 0)
            gather(g, 1, 1)
            if g + 1 < ng:
                idx_load(g + 1)

            def inner(jj2, carry):
                jj = jj2 * 2
                gather_wait(g, jj, 0)
                scatter(g, jj, 0)
                gather_wait(g, jj + 1, 1)
                scatter(g, jj + 1, 1)
                scatter_wait(g, jj, 0)
                gather(g, jj + 2, 0)
                scatter_wait(g, jj + 1, 1)
                gather(g, jj + 3, 1)
                return carry

            lax.fori_loop(0, IG // 2 - 1, inner, 0)
            gather_wait(g, IG - 2, 0)
            scatter(g, IG - 2, 0)
            gather_wait(g, IG - 1, 1)
            scatter(g, IG - 1, 1)
            scatter_wait(g, IG - 2, 0)
            scatter_wait(g, IG - 1, 1)
            if g + 1 < ng:
                idx_wait(g + 1)

    pipeline(NCHX // IG)

    plsc.subcore_barrier()
    pltpu.sync_copy(acc.at[pl.ds(s * RPT, RPT)],
                    out_hbm.at[c].at[pl.ds(s * RPT, RPT)])


def _l1_body(x_ref, w1_ref, d0_ref, d1_ref, h1p_ref, dis_ref):
    deg = d0_ref[...][:, :1] + d1_ref[...][:, :1] + 1.0
    dis = lax.rsqrt(deg)
    dis_ref[...] = dis
    h = jnp.dot(x_ref[...], w1_ref[...], preferred_element_type=jnp.float32)
    h1p_ref[...] = h * dis


def _l2_body(p0_ref, p1_ref, h1p_ref, dis_ref, b1_ref, w2_ref, h2p_ref):
    dis = dis_ref[...]
    t = (p0_ref[...] + p1_ref[...] + h1p_ref[...]) * dis + b1_ref[...]
    h = jnp.maximum(t, 0.0)
    h2p_ref[...] = jnp.dot(h, w2_ref[...],
                           preferred_element_type=jnp.float32) * dis


def _out_body(q0_ref, q1_ref, h2p_ref, dis_ref, b2_ref, out_ref):
    t = (q0_ref[...] + q1_ref[...] + h2p_ref[...]) * dis_ref[...] + b2_ref[...]
    m = jnp.max(t, axis=1, keepdims=True)
    lse = jnp.log(jnp.sum(jnp.exp(t - m), axis=1, keepdims=True)) + m
    out_ref[...] = t - lse


_row_spec = pl.BlockSpec((RB, D), lambda i: (i, 0))
_col_spec = pl.BlockSpec((RB, 1), lambda i: (i, 0))
_deg_spec = pl.BlockSpec((RB, D), lambda i: (i, 0))
_w_spec = pl.BlockSpec((D, D), lambda i: (0, 0))
_b_spec = pl.BlockSpec((1, D), lambda i: (0, 0))

_l1_call = pl.pallas_call(
    _l1_body,
    grid=(N // RB,),
    in_specs=[_row_spec, _w_spec, _deg_spec, _deg_spec],
    out_specs=[_row_spec, _col_spec],
    out_shape=[jax.ShapeDtypeStruct((N, D), jnp.float32),
               jax.ShapeDtypeStruct((N, 1), jnp.float32)],
)

_l2_call = pl.pallas_call(
    _l2_body,
    grid=(N // RB,),
    in_specs=[_row_spec, _row_spec, _row_spec, _col_spec, _b_spec, _w_spec],
    out_specs=_row_spec,
    out_shape=jax.ShapeDtypeStruct((N, D), jnp.float32),
)

_out_call = pl.pallas_call(
    _out_body,
    grid=(N // RB,),
    in_specs=[_row_spec, _row_spec, _row_spec, _col_spec, _b_spec],
    out_specs=_row_spec,
    out_shape=jax.ShapeDtypeStruct((N, D), jnp.float32),
)


def kernel(x, edge_index, W1, b1, W2, b2):
    src = edge_index[0].astype(jnp.int32)
    dst = edge_index[1].astype(jnp.int32)
    pad = E_PAD - E
    pad_src = jnp.zeros((pad,), jnp.int32)
    pad_dst = N + (jnp.arange(pad, dtype=jnp.int32) % (NACC - N))
    srcp = jnp.concatenate([src, pad_src])
    dstp = jnp.concatenate([dst, pad_dst])
    dst3 = dstp.reshape(NW, NCH, CH)

    E0 = NS * NCH0 * CH

    def msg_layout(a):
        a0 = a[:E0].reshape(NS, NCH0, CH)
        a0 = jnp.pad(a0, ((0, 0), (0, NCHX - NCH0), (0, 0)))
        a1 = a[E0:].reshape(NS, NCH1, CH)
        a1 = jnp.pad(a1, ((0, 0), (0, NCHX - NCH1), (0, 0)))
        return jnp.concatenate([a0, a1], axis=0)

    srcm = msg_layout(srcp)
    dstm = msg_layout(dstp)
    onesD = jnp.ones((CH, D), jnp.float32)
    zerosD = jnp.zeros((RPT, D), jnp.float32)

    degD = _deg_kernel(dst3, onesD, zerosD)
    h1p, dis = _l1_call(x, W1, degD[0, :N], degD[1, :N])
    p = _msg_kernel(h1p, srcm, dstm, zerosD)
    h2p = _l2_call(p[0, :N], p[1, :N], h1p, dis, b1.reshape(1, D), W2)
    q = _msg_kernel(h2p, srcm, dstm, zerosD)
    return _out_call(q[0, :N], q[1, :N], h2p, dis, b2.reshape(1, D))


# speedup vs baseline: 1.1864x; 1.0161x over previous
import functools

import jax
import jax.numpy as jnp
from jax import lax
from jax.experimental import pallas as pl
from jax.experimental.pallas import tpu as pltpu
from jax.experimental.pallas import tpu_sc as plsc

N = 10000
D = 128
E = 320000
NC = 2
NS = 16
NW = NC * NS
CH = 128
NCH = 80
EPT = CH * NCH
E_PAD = NW * EPT
NACC = 10240
RPT = NACC // NS
RB = 1000
IG = 16

_mesh = plsc.VectorSubcoreMesh(core_axis_name="c", subcore_axis_name="s")


@functools.partial(
    pl.kernel,
    out_type=jax.ShapeDtypeStruct((NC, NACC, D), jnp.float32),
    mesh=_mesh,
    scratch_types=[
        pltpu.VMEM((NCH, CH), jnp.int32),
        pltpu.VMEM((CH, D), jnp.float32),
        pltpu.VMEM_SHARED((NACC, D), jnp.float32),
        pltpu.SemaphoreType.DMA,
    ],
)
def _deg_kernel(dst_hbm, ones_hbm, zeros_hbm, out_hbm, dst_v, ones_v, acc, sem):
    c = lax.axis_index("c")
    s = lax.axis_index("s")
    wid = c * NS + s
    pltpu.sync_copy(zeros_hbm, acc.at[pl.ds(s * RPT, RPT)])
    pltpu.sync_copy(ones_hbm, ones_v)
    pltpu.sync_copy(dst_hbm.at[wid], dst_v)
    plsc.subcore_barrier()

    def body(j, carry):
        pltpu.sync_copy(ones_v, acc.at[dst_v.at[j]], add=True)
        return carry

    lax.fori_loop(0, NCH, body, 0)
    plsc.subcore_barrier()
    pltpu.sync_copy(acc.at[pl.ds(s * RPT, RPT)],
                    out_hbm.at[c].at[pl.ds(s * RPT, RPT)])


@functools.partial(
    pl.kernel,
    out_type=jax.ShapeDtypeStruct((NC, NACC, D), jnp.float32),
    mesh=_mesh,
    scratch_types=[
        pltpu.VMEM((NCH, CH), jnp.int32),
    ] + [pltpu.VMEM((IG, CH), jnp.int32)] * 2
      + [pltpu.VMEM((CH, D), jnp.float32)] * 2
      + [pltpu.VMEM_SHARED((NACC, D), jnp.float32)]
      + [pltpu.SemaphoreType.DMA] * 6,
)
def _msg_kernel(h_hbm, src_hbm, dst_hbm, zeros_hbm, out_hbm,
                dst_v, sidx0, sidx1, buf0, buf1, acc,
                isem0, isem1, gsem0, gsem1, ssem0, ssem1):
    sidxs = (sidx0, sidx1)
    bufs = (buf0, buf1)
    isems = (isem0, isem1)
    gsems = (gsem0, gsem1)
    ssems = (ssem0, ssem1)
    c = lax.axis_index("c")
    s = lax.axis_index("s")
    wid = c * NS + s
    NG = NCH // IG

    def idx_load(g):
        sl = g % 2
        pltpu.async_copy(src_hbm.at[wid].at[pl.ds(g * IG, IG)],
                         sidxs[sl], isems[sl])

    def idx_wait(g):
        sl = g % 2
        pltpu.make_async_copy(src_hbm.at[wid].at[pl.ds(g * IG, IG)],
                              sidxs[sl], isems[sl]).wait()

    def gather(g, jj, b):
        pltpu.async_copy(h_hbm.at[sidxs[g % 2].at[jj]], bufs[b], gsems[b])

    def gather_wait(g, jj, b):
        pltpu.make_async_copy(h_hbm.at[sidxs[g % 2].at[jj]],
                              bufs[b], gsems[b]).wait()

    def scatter(j, b):
        pltpu.async_copy(bufs[b], acc.at[dst_v.at[j]], ssems[b], add=True)

    def scatter_wait(j, b):
        pltpu.make_async_copy(bufs[b], acc.at[dst_v.at[j]], ssems[b]).wait()

    pltpu.sync_copy(zeros_hbm, acc.at[pl.ds(s * RPT, RPT)])
    pltpu.sync_copy(dst_hbm.at[wid], dst_v)
    idx_load(0)
    plsc.subcore_barrier()
    idx_wait(0)

    for g in range(NG):
        base = g * IG
        gather(g, 0, 0)
        gather(g, 1, 1)
        if g + 1 < NG:
            idx_load(g + 1)

        def inner(jj2, carry):
            jj = jj2 * 2
            gather_wait(g, jj, 0)
            scatter(base + jj, 0)
            gather_wait(g, jj + 1, 1)
            scatter(base + jj + 1, 1)
            scatter_wait(base + jj, 0)
            gather(g, jj + 2, 0)
            scatter_wait(base + jj + 1, 1)
            gather(g, jj + 3, 1)
            return carry

        lax.fori_loop(0, IG // 2 - 1, inner, 0)
        gather_wait(g, IG - 2, 0)
        scatter(base + IG - 2, 0)
        gather_wait(g, IG - 1, 1)
        scatter(base + IG - 1, 1)
        scatter_wait(base + IG - 2, 0)
        scatter_wait(base + IG - 1, 1)
        if g + 1 < NG:
            idx_wait(g + 1)

    plsc.subcore_barrier()
    pltpu.sync_copy(acc.at[pl.ds(s * RPT, RPT)],
                    out_hbm.at[c].at[pl.ds(s * RPT, RPT)])


def _l1_body(x_ref, w1_ref, d0_ref, d1_ref, h1p_ref, dis_ref):
    deg = d0_ref[...][:, :1] + d1_ref[...][:, :1] + 1.0
    dis = lax.rsqrt(deg)
    dis_ref[...] = dis
    h = jnp.dot(x_ref[...], w1_ref[...], preferred_element_type=jnp.float32)
    h1p_ref[...] = h * dis


def _l2_body(p0_ref, p1_ref, h1p_ref, dis_ref, b1_ref, w2_ref, h2p_ref):
    dis = dis_ref[...]
    t = (p0_ref[...] + p1_ref[...] + h1p_ref[...]) * dis + b1_ref[...]
    h = jnp.maximum(t, 0.0)
    h2p_ref[...] = jnp.dot(h, w2_ref[...],
                           preferred_element_type=jnp.float32) * dis


def _out_body(q0_ref, q1_ref, h2p_ref, dis_ref, b2_ref, out_ref):
    t = (q0_ref[...] + q1_ref[...] + h2p_ref[...]) * dis_ref[...] + b2_ref[...]
    m = jnp.max(t, axis=1, keepdims=True)
    lse = jnp.log(jnp.sum(jnp.exp(t - m), axis=1, keepdims=True)) + m
    out_ref[...] = t - lse


_row_spec = pl.BlockSpec((RB, D), lambda i: (i, 0))
_col_spec = pl.BlockSpec((RB, 1), lambda i: (i, 0))
_deg_spec = pl.BlockSpec((RB, D), lambda i: (i, 0))
_w_spec = pl.BlockSpec((D, D), lambda i: (0, 0))
_b_spec = pl.BlockSpec((1, D), lambda i: (0, 0))

_l1_call = pl.pallas_call(
    _l1_body,
    grid=(N // RB,),
    in_specs=[_row_spec, _w_spec, _deg_spec, _deg_spec],
    out_specs=[_row_spec, _col_spec],
    out_shape=[jax.ShapeDtypeStruct((N, D), jnp.float32),
               jax.ShapeDtypeStruct((N, 1), jnp.float32)],
)

_l2_call = pl.pallas_call(
    _l2_body,
    grid=(N // RB,),
    in_specs=[_row_spec, _row_spec, _row_spec, _col_spec, _b_spec, _w_spec],
    out_specs=_row_spec,
    out_shape=jax.ShapeDtypeStruct((N, D), jnp.float32),
)

_out_call = pl.pallas_call(
    _out_body,
    grid=(N // RB,),
    in_specs=[_row_spec, _row_spec, _row_spec, _col_spec, _b_spec],
    out_specs=_row_spec,
    out_shape=jax.ShapeDtypeStruct((N, D), jnp.float32),
)


def kernel(x, edge_index, W1, b1, W2, b2):
    src = edge_index[0].astype(jnp.int32)
    dst = edge_index[1].astype(jnp.int32)
    pad = E_PAD - E
    pad_src = jnp.zeros((pad,), jnp.int32)
    pad_dst = N + (jnp.arange(pad, dtype=jnp.int32) % (NACC - N))
    srcm = jnp.concatenate([src, pad_src]).reshape(NW, NCH, CH)
    dstm = jnp.concatenate([dst, pad_dst]).reshape(NW, NCH, CH)
    dst3 = dstm
    onesD = jnp.ones((CH, D), jnp.float32)
    zerosD = jnp.zeros((RPT, D), jnp.float32)

    degD = _deg_kernel(dst3, onesD, zerosD)
    h1p, dis = _l1_call(x, W1, degD[0, :N], degD[1, :N])
    p = _msg_kernel(h1p, srcm, dstm, zerosD)
    h2p = _l2_call(p[0, :N], p[1, :N], h1p, dis, b1.reshape(1, D), W2)
    q = _msg_kernel(h2p, srcm, dstm, zerosD)
    return _out_call(q[0, :N], q[1, :N], h2p, dis, b2.reshape(1, D))
